# baseline (speedup 1.0000x reference)
import functools

import jax
import jax.numpy as jnp
from jax import lax
from jax.experimental import pallas as pl
from jax.experimental.pallas import tpu as pltpu
from jax.experimental.pallas import tpu_sc as plsc

_B = 16384
_D = 32
_V = 100000
_VP = _V * _D // 128
_NP = 4
_RB = 4
_OB = 2


_NREP = 25


def _tc_repack_body(cemb_ref, packed_ref):
    x = cemb_ref[...].reshape(_V // _NREP // 4, 4, _D)
    for u in range(4):
        packed_ref[:, u * _D:(u + 1) * _D] = x[:, u, :]


@functools.cache
def _build_repack():
    return pl.pallas_call(
        _tc_repack_body,
        grid=(_NREP,),
        in_specs=[pl.BlockSpec((_V // _NREP, _D), lambda i: (i, 0))],
        out_specs=pl.BlockSpec((_VP // _NREP, 128), lambda i: (i, 0)),
        out_shape=jax.ShapeDtypeStruct((_VP, 128), jnp.float32),
    )


@functools.cache
def _build():
    info = plsc.get_sparse_core_info()
    nw = info.num_cores * info.num_subcores
    bpw = _B // nw
    nc = info.num_cores
    chunk = bpw // _NP
    mesh = plsc.VectorSubcoreMesh(core_axis_name="c", subcore_axis_name="s")

    @functools.partial(
        pl.kernel,
        mesh=mesh,
        compiler_params=pltpu.CompilerParams(use_tc_tiling_on_sc=True,
                                             needs_layout_passes=False),
        out_type=jax.ShapeDtypeStruct((_B, 2 * _D), jnp.float32),
        scratch_types=[
            pltpu.VMEM((bpw,), jnp.int32),
            pltpu.VMEM((bpw,), jnp.int32),
            pltpu.VMEM((bpw,), jnp.int32),
            pltpu.VMEM((8, _D), jnp.float32),
            *[pltpu.VMEM((chunk, 128), jnp.float32) for _ in range(_RB)],
            *[pltpu.VMEM((chunk, 2 * _D), jnp.float32) for _ in range(_OB)],
            *[pltpu.SemaphoreType.DMA for _ in range(_RB + _OB)],
        ],
    )
    def lookup_kernel(sid_hbm, cid_hbm, semb_hbm, packed_hbm, out_hbm,
                      sidx_v, cidx_v, ci4_v, stab_v, *bufs):
        crows = bufs[:_RB]
        outs = bufs[_RB:_RB + _OB]
        gsems = bufs[_RB + _OB:2 * _RB + _OB]
        osems = bufs[2 * _RB + _OB:]
        wid = lax.axis_index("s") * nc + lax.axis_index("c")
        base = wid * bpw
        pltpu.sync_copy(sid_hbm.at[pl.ds(base, bpw)], sidx_v)
        pltpu.sync_copy(cid_hbm.at[pl.ds(base, bpw)], cidx_v)
        pltpu.sync_copy(semb_hbm, stab_v)
        for g in range(bpw // 16):
            sl = pl.ds(g * 16, 16)
            ci4_v[sl] = lax.shift_right_logical(cidx_v[sl], 2)

        lanes = lax.iota(jnp.int32, 16)

        def fire_gather(p):
            return pltpu.async_copy(
                packed_hbm.at[ci4_v.at[pl.ds(p * chunk, chunk)]],
                crows[p % _RB].at[:], gsems[p % _RB])

        def make_assemble(p):
            crow_v, out_v = crows[p % _RB], outs[p % _OB]

            def assemble(g, _):
                loc16 = g * 16 + lanes
                rows16 = p * chunk + loc16
                sid16 = plsc.load_gather(sidx_v, [rows16])
                cid16 = plsc.load_gather(cidx_v, [rows16])
                ccol = (cid16 & 3) * _D
                for d in range(_D):
                    dskew = (lanes + d) & (_D - 1)
                    sval = plsc.load_gather(stab_v, [sid16, dskew])
                    plsc.store_scatter(out_v, [loc16, dskew], sval)
                    cval = plsc.load_gather(crow_v, [loc16, ccol + dskew])
                    plsc.store_scatter(out_v, [loc16, dskew + _D], cval)
                return _
            return assemble

        gcps = {p: fire_gather(p) for p in range(_RB)}
        ocps = {}
        for p in range(_NP):
            gcps[p].wait()
            if p - _OB in ocps:
                ocps[p - _OB].wait()
            lax.fori_loop(0, chunk // 16, make_assemble(p), 0)
            ocps[p] = pltpu.async_copy(
                outs[p % _OB], out_hbm.at[pl.ds(base + p * chunk, chunk)],
                osems[p % _OB])
            if p + _RB < _NP:
                gcps[p + _RB] = fire_gather(p + _RB)
        for p in range(max(0, _NP - _OB), _NP):
            ocps[p].wait()

    return lookup_kernel


def kernel(strategy_id, cause_index, strategy_emb, cause_emb):
    packed = _build_repack()(cause_emb)
    return _build()(strategy_id.astype(jnp.int32),
                    cause_index.astype(jnp.int32),
                    strategy_emb, packed)


# speedup vs baseline: 1.1595x; 1.1595x over previous
import functools

import jax
import jax.numpy as jnp
from jax import lax
from jax.experimental import pallas as pl
from jax.experimental.pallas import tpu as pltpu
from jax.experimental.pallas import tpu_sc as plsc

_B = 16384
_D = 32
_V = 100000
_VP = _V * _D // 128
_NP = 4
_RB = 4
_OB = 2


@functools.cache
def _build():
    info = plsc.get_sparse_core_info()
    nw = info.num_cores * info.num_subcores
    bpw = _B // nw
    nc = info.num_cores
    chunk = bpw // _NP
    mesh = plsc.VectorSubcoreMesh(core_axis_name="c", subcore_axis_name="s")

    @functools.partial(
        pl.kernel,
        mesh=mesh,
        compiler_params=pltpu.CompilerParams(use_tc_tiling_on_sc=True,
                                             needs_layout_passes=False),
        out_type=jax.ShapeDtypeStruct((_B, 2 * _D), jnp.float32),
        scratch_types=[
            pltpu.VMEM((bpw,), jnp.int32),
            pltpu.VMEM((bpw,), jnp.int32),
            pltpu.VMEM((bpw,), jnp.int32),
            pltpu.VMEM((8, _D), jnp.float32),
            *[pltpu.VMEM((chunk, 128), jnp.float32) for _ in range(_RB)],
            *[pltpu.VMEM((chunk, 2 * _D), jnp.float32) for _ in range(_OB)],
            *[pltpu.SemaphoreType.DMA for _ in range(_RB + _OB)],
        ],
    )
    def lookup_kernel(sid_hbm, cid_hbm, semb_hbm, packed_hbm, out_hbm,
                      sidx_v, cidx_v, ci4_v, stab_v, *bufs):
        crows = bufs[:_RB]
        outs = bufs[_RB:_RB + _OB]
        gsems = bufs[_RB + _OB:2 * _RB + _OB]
        osems = bufs[2 * _RB + _OB:]
        wid = lax.axis_index("s") * nc + lax.axis_index("c")
        base = wid * bpw
        pltpu.sync_copy(sid_hbm.at[pl.ds(base, bpw)], sidx_v)
        pltpu.sync_copy(cid_hbm.at[pl.ds(base, bpw)], cidx_v)
        pltpu.sync_copy(semb_hbm, stab_v)
        for g in range(bpw // 16):
            sl = pl.ds(g * 16, 16)
            ci4_v[sl] = lax.shift_right_logical(cidx_v[sl], 2)

        lanes = lax.iota(jnp.int32, 16)

        def fire_gather(p):
            return pltpu.async_copy(
                packed_hbm.at[ci4_v.at[pl.ds(p * chunk, chunk)]],
                crows[p % _RB].at[:], gsems[p % _RB])

        def make_strategy(p):
            out_v = outs[p % _OB]

            def strat(g, _):
                loc16 = g * 16 + lanes
                sid16 = plsc.load_gather(sidx_v, [p * chunk + loc16])
                for d in range(_D):
                    dskew = (lanes + d) & (_D - 1)
                    sval = plsc.load_gather(stab_v, [sid16, dskew])
                    plsc.store_scatter(out_v, [loc16, dskew], sval)
                return _
            return strat

        def make_cause(p):
            crow_v, out_v = crows[p % _RB], outs[p % _OB]

            def cause(g, _):
                loc16 = g * 16 + lanes
                cid16 = plsc.load_gather(cidx_v, [p * chunk + loc16])
                ccol = (cid16 & 3) * _D
                for d in range(_D):
                    dskew = (lanes + d) & (_D - 1)
                    cval = plsc.load_gather(crow_v, [loc16, ccol + dskew])
                    plsc.store_scatter(out_v, [loc16, dskew + _D], cval)
                return _
            return cause

        gcps = {p: fire_gather(p) for p in range(_RB)}
        ocps = {}
        for p in range(_NP):
            if p - _OB in ocps:
                ocps[p - _OB].wait()
            lax.fori_loop(0, chunk // 16, make_strategy(p), 0)
            gcps[p].wait()
            lax.fori_loop(0, chunk // 16, make_cause(p), 0)
            ocps[p] = pltpu.async_copy(
                outs[p % _OB], out_hbm.at[pl.ds(base + p * chunk, chunk)],
                osems[p % _OB])
            if p + _RB < _NP:
                gcps[p + _RB] = fire_gather(p + _RB)
        for p in range(max(0, _NP - _OB), _NP):
            ocps[p].wait()

    return lookup_kernel


def kernel(strategy_id, cause_index, strategy_emb, cause_emb):
    packed = cause_emb.reshape(_VP, 128)
    return _build()(strategy_id.astype(jnp.int32),
                    cause_index.astype(jnp.int32),
                    strategy_emb, packed)


# speedup vs baseline: 1.1828x; 1.0201x over previous
import functools

import jax
import jax.numpy as jnp
from jax import lax
from jax.experimental import pallas as pl
from jax.experimental.pallas import tpu as pltpu
from jax.experimental.pallas import tpu_sc as plsc

_B = 16384
_D = 32
_V = 100000
_VP = _V * _D // 128
_NP = 4
_RB = 4
_OB = 2


@functools.cache
def _build():
    info = plsc.get_sparse_core_info()
    nw = info.num_cores * info.num_subcores
    bpw = _B // nw
    nc = info.num_cores
    chunk = bpw // _NP
    mesh = plsc.VectorSubcoreMesh(core_axis_name="c", subcore_axis_name="s")

    @functools.partial(
        pl.kernel,
        mesh=mesh,
        compiler_params=pltpu.CompilerParams(use_tc_tiling_on_sc=True,
                                             needs_layout_passes=False),
        out_type=jax.ShapeDtypeStruct((_B, 2 * _D), jnp.float32),
        scratch_types=[
            pltpu.VMEM((bpw,), jnp.int32),
            pltpu.VMEM((bpw,), jnp.int32),
            pltpu.VMEM((bpw,), jnp.int32),
            pltpu.VMEM((8, _D), jnp.float32),
            *[pltpu.VMEM((chunk, 128), jnp.float32) for _ in range(_RB)],
            *[pltpu.VMEM((chunk, 2 * _D), jnp.float32) for _ in range(_OB)],
            *[pltpu.SemaphoreType.DMA for _ in range(_RB + _OB)],
        ],
    )
    def lookup_kernel(sid_hbm, cid_hbm, semb_hbm, packed_hbm, out_hbm,
                      sidx_v, cidx_v, ci4_v, stab_v, *bufs):
        crows = bufs[:_RB]
        outs = bufs[_RB:_RB + _OB]
        gsems = bufs[_RB + _OB:2 * _RB + _OB]
        osems = bufs[2 * _RB + _OB:]
        wid = lax.axis_index("s") * nc + lax.axis_index("c")
        base = wid * bpw
        pltpu.sync_copy(sid_hbm.at[pl.ds(base, bpw)], sidx_v)
        pltpu.sync_copy(cid_hbm.at[pl.ds(base, bpw)], cidx_v)
        pltpu.sync_copy(semb_hbm, stab_v)
        for g in range(bpw // 16):
            sl = pl.ds(g * 16, 16)
            ci4_v[sl] = lax.shift_right_logical(cidx_v[sl], 2)

        lanes = lax.iota(jnp.int32, 16)

        def fire_gather(p):
            return pltpu.async_copy(
                packed_hbm.at[ci4_v.at[pl.ds(p * chunk, chunk)]],
                crows[p % _RB].at[:], gsems[p % _RB])

        def make_assemble(p):
            crow_v, out_v = crows[p % _RB], outs[p % _OB]

            def assemble(g, _):
                loc16 = g * 16 + lanes
                rows16 = p * chunk + loc16
                sid16 = plsc.load_gather(sidx_v, [rows16])
                cid16 = plsc.load_gather(cidx_v, [rows16])
                ccol = (cid16 & 3) * _D
                for d in range(_D):
                    dskew = (lanes + d) & (_D - 1)
                    sval = plsc.load_gather(stab_v, [sid16, dskew])
                    plsc.store_scatter(out_v, [loc16, dskew], sval)
                    cval = plsc.load_gather(crow_v, [loc16, ccol + dskew])
                    plsc.store_scatter(out_v, [loc16, dskew + _D], cval)
                return _
            return assemble

        gcps = {p: fire_gather(p) for p in range(_RB)}
        ocps = {}
        for p in range(_NP):
            gcps[p].wait()
            if p - _OB in ocps:
                ocps[p - _OB].wait()
            lax.fori_loop(0, chunk // 16, make_assemble(p), 0)
            ocps[p] = pltpu.async_copy(
                outs[p % _OB], out_hbm.at[pl.ds(base + p * chunk, chunk)],
                osems[p % _OB])
            if p + _RB < _NP:
                gcps[p + _RB] = fire_gather(p + _RB)
        for p in range(max(0, _NP - _OB), _NP):
            ocps[p].wait()

    return lookup_kernel


def kernel(strategy_id, cause_index, strategy_emb, cause_emb):
    packed = cause_emb.reshape(_VP, 128)
    return _build()(strategy_id.astype(jnp.int32),
                    cause_index.astype(jnp.int32),
                    strategy_emb, packed)
